# baseline (speedup 1.0000x reference)
import functools

import jax
import jax.numpy as jnp
from jax import lax
from jax.experimental import pallas as pl
from jax.experimental.pallas import tpu as pltpu
from jax.experimental.pallas import tpu_sc as plsc

E = 800000
D = 64
NC = 2
NS = 16
NW = NC * NS
L = 16
SUP = 1280
HALF = SUP // 2
GPH = HALF // L
NSUP = E // SUP
KMAX = (NSUP + NW - 1) // NW

_mesh = plsc.VectorSubcoreMesh(core_axis_name="c", subcore_axis_name="s")


@functools.partial(
    pl.kernel,
    out_type=jax.ShapeDtypeStruct((E, D), jnp.float32),
    mesh=_mesh,
    compiler_params=pltpu.CompilerParams(use_tc_tiling_on_sc=False,
                                         needs_layout_passes=False),
    scratch_types=[
        pltpu.VMEM((60 * D,), jnp.float32),
        pltpu.VMEM((3 * SUP,), jnp.int32),
        pltpu.VMEM((HALF, D), jnp.float32),
        pltpu.VMEM((HALF, D), jnp.float32),
        pltpu.SemaphoreType.DMA((2,)),
    ],
)
def _sc_lookup(attr_h, tab_h, out_h, tab_v, attr_v, rows0_v, rows1_v, ssem):
    wid = lax.axis_index("s") * NC + lax.axis_index("c")
    pltpu.sync_copy(tab_h, tab_v)
    iota = lax.iota(jnp.int32, L)
    iota3 = iota * 3

    def do_half(rows_v, gofs, _):
        def group(g, carry):
            pos = iota3 + (3 * L) * (g + gofs)
            a0 = plsc.load_gather(attr_v, [pos])
            a1 = plsc.load_gather(attr_v, [pos + 1])
            a2 = plsc.load_gather(attr_v, [pos + 2])
            r64 = a0 * (12 * D) + a1 * (2 * D) + a2 * D
            rvec = iota + g * L
            for j in range(D):
                v = plsc.load_gather(tab_v, [r64 + j])
                plsc.store_scatter(rows_v, [rvec, jnp.full((L,), j,
                                                           jnp.int32)], v)
            return carry

        lax.fori_loop(0, GPH, group, 0)

    def step(k, carry):
        sup = k * NW + wid

        @pl.when(sup < NSUP)
        def _():
            ebase = sup * SUP
            pltpu.sync_copy(attr_h.at[pl.ds(3 * ebase, 3 * SUP)], attr_v)

            @pl.when(k > 0)
            def _():
                pltpu.make_async_copy(
                    out_h.at[pl.ds(0, HALF)], rows0_v, ssem.at[0]).wait()

            do_half(rows0_v, 0, None)
            pltpu.async_copy(rows0_v, out_h.at[pl.ds(ebase, HALF)],
                             ssem.at[0])

            @pl.when(k > 0)
            def _():
                pltpu.make_async_copy(
                    out_h.at[pl.ds(0, HALF)], rows1_v, ssem.at[1]).wait()

            do_half(rows1_v, GPH, None)
            pltpu.async_copy(rows1_v, out_h.at[pl.ds(ebase + HALF, HALF)],
                             ssem.at[1])

        return carry

    lax.fori_loop(0, KMAX, step, 0)
    pltpu.make_async_copy(out_h.at[pl.ds(0, HALF)], rows0_v,
                          ssem.at[0]).wait()
    pltpu.make_async_copy(out_h.at[pl.ds(0, HALF)], rows1_v,
                          ssem.at[1]).wait()


def kernel(edge_attr, W0, W1, W2):
    attr_flat = edge_attr.astype(jnp.int32).reshape(3 * E)
    tab = (W0[:, None, None, :] + W1[None, :, None, :]
           + W2[None, None, :, :]).reshape(60 * D)
    return _sc_lookup(attr_flat, tab)


# speedup vs baseline: 1.5175x; 1.5175x over previous
import functools

import jax
import jax.numpy as jnp
from jax import lax
from jax.experimental import pallas as pl
from jax.experimental.pallas import tpu as pltpu
from jax.experimental.pallas import tpu_sc as plsc

E = 800000
D = 64
NC = 2
NS = 16
NW = NC * NS
L = 16
SUP = 6400
NSUP = E // SUP
KMAX = (NSUP + NW - 1) // NW
BLK = 4000
NBLK = E // BLK

_mesh = plsc.VectorSubcoreMesh(core_axis_name="c", subcore_axis_name="s")


@functools.partial(
    pl.kernel,
    out_type=jax.ShapeDtypeStruct((E,), jnp.int32),
    mesh=_mesh,
    compiler_params=pltpu.CompilerParams(use_tc_tiling_on_sc=False,
                                         needs_layout_passes=False),
    scratch_types=[
        pltpu.VMEM((3 * SUP,), jnp.int32),
        pltpu.VMEM((SUP,), jnp.int32),
    ],
)
def _sc_indices(attr_h, idx_h, attr_v, idx_v):
    wid = lax.axis_index("s") * NC + lax.axis_index("c")
    iota3 = lax.iota(jnp.int32, L) * 3

    for k in range(KMAX):
        sup = k * NW + wid

        @pl.when(sup < NSUP)
        def _():
            ebase = sup * SUP
            pltpu.sync_copy(attr_h.at[pl.ds(3 * ebase, 3 * SUP)], attr_v)

            def group(g, carry):
                pos = iota3 + (3 * L) * g
                a0 = plsc.load_gather(attr_v, [pos])
                a1 = plsc.load_gather(attr_v, [pos + 1])
                a2 = plsc.load_gather(attr_v, [pos + 2])
                idx_v[pl.ds(g * L, L)] = a0 * 12 + a1 * 2 + a2
                return carry

            lax.fori_loop(0, SUP // L, group, 0)
            pltpu.sync_copy(idx_v, idx_h.at[pl.ds(ebase, SUP)])


def _tc_expand_body(idx_ref, tab_ref, out_ref):
    idxs = idx_ref[0, 0, :]
    cols = lax.broadcasted_iota(jnp.int32, (BLK, D), 1)
    onehot = (idxs[:, None] == cols).astype(jnp.float32)
    out_ref[...] = lax.dot_general(
        onehot, tab_ref[...], (((1,), (0,)), ((), ())),
        precision=lax.Precision.HIGHEST,
        preferred_element_type=jnp.float32)


_tc_expand = pl.pallas_call(
    _tc_expand_body,
    grid=(NBLK,),
    in_specs=[
        pl.BlockSpec((1, 1, BLK), lambda i: (i, 0, 0)),
        pl.BlockSpec((D, D), lambda i: (0, 0)),
    ],
    out_specs=pl.BlockSpec((BLK, D), lambda i: (i, 0)),
    out_shape=jax.ShapeDtypeStruct((E, D), jnp.float32),
    compiler_params=pltpu.CompilerParams(
        dimension_semantics=("arbitrary",)),
)


def kernel(edge_attr, W0, W1, W2):
    attr_flat = edge_attr.astype(jnp.int32).reshape(3 * E)
    tab = (W0[:, None, None, :] + W1[None, :, None, :]
           + W2[None, None, :, :]).reshape(60, D)
    tab = jnp.pad(tab, ((0, D - 60), (0, 0)))
    idx = _sc_indices(attr_flat)
    return _tc_expand(idx.reshape(NBLK, 1, BLK), tab)


# speedup vs baseline: 7.2202x; 4.7579x over previous
import functools

import jax
import jax.numpy as jnp
from jax import lax
from jax.experimental import pallas as pl
from jax.experimental.pallas import tpu as pltpu
from jax.experimental.pallas import tpu_sc as plsc

E = 800000
D = 64
NC = 2
NS = 16
NW = NC * NS
L = 16
SUP = 6400
NSUP = E // SUP
KMAX = (NSUP + NW - 1) // NW
BLK = 8192
NBLK = -(-E // BLK)
EP = NBLK * BLK

_mesh = plsc.VectorSubcoreMesh(core_axis_name="c", subcore_axis_name="s")


@functools.partial(
    pl.kernel,
    out_type=jax.ShapeDtypeStruct((EP,), jnp.int32),
    mesh=_mesh,
    compiler_params=pltpu.CompilerParams(use_tc_tiling_on_sc=False,
                                         needs_layout_passes=False),
    scratch_types=[
        pltpu.VMEM((SUP,), jnp.int32),
        pltpu.VMEM((SUP,), jnp.int32),
        pltpu.VMEM((SUP,), jnp.int32),
        pltpu.VMEM((SUP,), jnp.int32),
    ],
)
def _sc_indices(a0_h, a1_h, a2_h, idx_h, a0_v, a1_v, a2_v, idx_v):
    wid = lax.axis_index("s") * NC + lax.axis_index("c")

    for k in range(KMAX):
        sup = k * NW + wid

        @pl.when(sup < NSUP)
        def _():
            ebase = sup * SUP
            pltpu.sync_copy(a0_h.at[pl.ds(ebase, SUP)], a0_v)
            pltpu.sync_copy(a1_h.at[pl.ds(ebase, SUP)], a1_v)
            pltpu.sync_copy(a2_h.at[pl.ds(ebase, SUP)], a2_v)

            def group(g, carry):
                s = pl.ds(g * L, L)
                idx_v[s] = a0_v[s] * 12 + a1_v[s] * 2 + a2_v[s]
                return carry

            lax.fori_loop(0, SUP // L, group, 0)
            pltpu.sync_copy(idx_v, idx_h.at[pl.ds(ebase, SUP)])


def _tc_expand_body(idx_ref, tab_ref, out_ref):
    idxs = idx_ref[...]
    rows = lax.broadcasted_iota(jnp.int32, (D, BLK), 0)
    oh_t = (rows == idxs[None, :]).astype(jnp.float32)
    out_ref[...] = lax.dot_general(
        oh_t, tab_ref[...], (((0,), (0,)), ((), ())),
        precision=lax.Precision.HIGHEST,
        preferred_element_type=jnp.float32)


_tc_expand = pl.pallas_call(
    _tc_expand_body,
    grid=(NBLK,),
    in_specs=[
        pl.BlockSpec((BLK,), lambda i: (i,)),
        pl.BlockSpec((D, D), lambda i: (0, 0)),
    ],
    out_specs=pl.BlockSpec((BLK, D), lambda i: (i, 0)),
    out_shape=jax.ShapeDtypeStruct((E, D), jnp.float32),
    compiler_params=pltpu.CompilerParams(
        dimension_semantics=("arbitrary",)),
)


def kernel(edge_attr, W0, W1, W2):
    ea = edge_attr.astype(jnp.int32)
    a0 = ea[:, 0]
    a1 = ea[:, 1]
    a2 = ea[:, 2]
    tab = (W0[:, None, None, :] + W1[None, :, None, :]
           + W2[None, None, :, :]).reshape(60, D)
    tab = jnp.pad(tab, ((0, D - 60), (0, 0)))
    idx = _sc_indices(a0, a1, a2)
    return _tc_expand(idx, tab)
